# Initial kernel scaffold; baseline (speedup 1.0000x reference)
import jax
import jax.numpy as jnp
from jax.experimental import pallas as pl


def kernel(rpn_cls_scores, rpn_bbox_adjusts, img_shapes, train):
    raise NotImplementedError("write your pallas kernel here")


# speedup vs baseline: 3.2591x; 3.2591x over previous
import functools

import jax
import jax.numpy as jnp
import numpy as np
from jax import lax
from jax.experimental import pallas as pl
from jax.experimental.pallas import tpu as pltpu

_INTERPRET = False

FEATURE_STRIDE = 16
PRE_NMS_TOP_N = 6000
POST_NMS_TOP_N = 300
NMS_THRESH = 0.7
MIN_SIZE = 16.0
BBOX_XFORM_CLIP = float(np.log(1000.0 / 16.0))
H = W = 64
A = 9
N = H * W * A
ROWS = N // 128
OUT_ROWS = 304


def _anchor_tables():
    base_size = float(FEATURE_STRIDE)
    x_ctr = (base_size - 1.0) * 0.5
    y_ctr = (base_size - 1.0) * 0.5
    size = base_size * base_size
    scales = np.array([8.0, 16.0, 32.0])
    ratios = np.array([0.5, 1.0, 2.0])
    size_ratios = size / ratios
    ws = np.round(np.sqrt(size_ratios))
    hs = np.round(ws * ratios)
    anchors = []
    for wr, hr in zip(ws, hs):
        for s in scales:
            aw = wr * s
            ah = hr * s
            anchors.append([x_ctr - 0.5 * (aw - 1.0), y_ctr - 0.5 * (ah - 1.0),
                            x_ctr + 0.5 * (aw - 1.0), y_ctr + 0.5 * (ah - 1.0)])
    base = np.asarray(anchors, dtype=np.float64)
    sx = np.arange(W, dtype=np.float64) * FEATURE_STRIDE
    sy = np.arange(H, dtype=np.float64) * FEATURE_STRIDE
    ys, xs = np.meshgrid(sy, sx, indexing="ij")
    shifts = np.stack([xs.ravel(), ys.ravel(), xs.ravel(), ys.ravel()], axis=1)
    anc = (base[None, :, :] + shifts[:, None, :]).reshape(-1, 4)
    wa = anc[:, 2] - anc[:, 0] + 1.0
    ha = anc[:, 3] - anc[:, 1] + 1.0
    cxa = anc[:, 0] + 0.5 * wa
    cya = anc[:, 1] + 0.5 * ha
    shape = (ROWS, 128)
    return (wa.astype(np.float32).reshape(shape), ha.astype(np.float32).reshape(shape),
            cxa.astype(np.float32).reshape(shape), cya.astype(np.float32).reshape(shape))


_WA, _HA, _CXA, _CYA = _anchor_tables()


def _proposal_body(img_ref, s_ref, dx_ref, dy_ref, dw_ref, dh_ref,
                   wa_ref, ha_ref, cxa_ref, cya_ref, out_ref,
                   x1s, y1s, x2s, y2s, areas, css):
    im_h = img_ref[0, 0, 0]
    im_w = img_ref[0, 0, 1]
    wa = wa_ref[...]
    ha = ha_ref[...]
    dw = jnp.clip(dw_ref[...], -BBOX_XFORM_CLIP, BBOX_XFORM_CLIP)
    dh = jnp.clip(dh_ref[...], -BBOX_XFORM_CLIP, BBOX_XFORM_CLIP)
    pw = jnp.exp(dw) * wa
    ph = jnp.exp(dh) * ha
    pcx = dx_ref[...] * wa + cxa_ref[...]
    pcy = dy_ref[...] * ha + cya_ref[...]
    x1 = jnp.clip(pcx - 0.5 * pw, 0.0, im_w - 1.0)
    y1 = jnp.clip(pcy - 0.5 * ph, 0.0, im_h - 1.0)
    x2 = jnp.clip(pcx + 0.5 * pw, 0.0, im_w - 1.0)
    y2 = jnp.clip(pcy + 0.5 * ph, 0.0, im_h - 1.0)
    ws = x2 - x1 + 1.0
    hs = y2 - y1 + 1.0
    s = jnp.where((ws >= MIN_SIZE) & (hs >= MIN_SIZE), s_ref[...], -1e10)

    y = lax.bitcast_convert_type(s, jnp.int32)
    key = y ^ ((y >> 31) & jnp.int32(0x7FFFFFFF))

    def tstep(i, p):
        cand = p + (jnp.int32(1) << (jnp.int32(31) - i))
        cnt = jnp.sum((key >= cand).astype(jnp.int32))
        return jnp.where(cnt >= PRE_NMS_TOP_N, cand, p)

    tau = lax.fori_loop(0, 32, tstep, jnp.int32(-(2 ** 31)))

    gt = key > tau
    eq = key == tau
    need = (PRE_NMS_TOP_N - jnp.sum(gt.astype(jnp.int32))).astype(jnp.float32)
    eqf = eq.astype(jnp.float32)
    mlane = (lax.broadcasted_iota(jnp.int32, (128, 128), 0)
             < lax.broadcasted_iota(jnp.int32, (128, 128), 1)).astype(jnp.float32)
    in_row = jnp.dot(eqf, mlane, preferred_element_type=jnp.float32)
    rowtot = jnp.broadcast_to(jnp.sum(eqf, axis=1, keepdims=True), (ROWS, 128))
    lrow = (lax.broadcasted_iota(jnp.int32, (ROWS, ROWS), 1)
            < lax.broadcasted_iota(jnp.int32, (ROWS, ROWS), 0)).astype(jnp.float32)
    rows_before = jnp.dot(lrow, rowtot, preferred_element_type=jnp.float32)
    sel = gt | (eq & (rows_before + in_row < need))
    cs = jnp.where(sel & (s > -1e9), s, -jnp.inf)

    x1s[...] = x1
    y1s[...] = y1
    x2s[...] = x2
    y2s[...] = y2
    areas[...] = ws * hs
    css[...] = cs

    bf = pl.program_id(0).astype(jnp.float32)
    li8 = lax.broadcasted_iota(jnp.int32, (OUT_ROWS, 8), 1)
    out_ref[...] = jnp.where(li8 == 0, bf, 0.0)

    rowi = lax.broadcasted_iota(jnp.int32, (ROWS, 128), 0)
    lanei = lax.broadcasted_iota(jnp.int32, (ROWS, 128), 1)
    flat = rowi * 128 + lanei
    lrow1 = lax.broadcasted_iota(jnp.int32, (1, 128), 1)
    li1 = lax.broadcasted_iota(jnp.int32, (1, 8), 1)

    def cond(carry):
        step, alive = carry
        return (step < POST_NMS_TOP_N) & alive

    def body(carry):
        step, _ = carry
        cs = css[...]
        m = jnp.max(cs)
        alive = m > -1e9

        @pl.when(alive)
        def _():
            j = jnp.min(jnp.where(cs == m, flat, jnp.int32(2 ** 30)))
            r = j >> 7
            c = j & 127

            def pick(ref):
                row = ref[pl.ds(r, 1), :]
                return jnp.sum(jnp.where(lrow1 == c, row, 0.0))

            x1j = pick(x1s)
            y1j = pick(y1s)
            x2j = pick(x2s)
            y2j = pick(y2s)
            aj = pick(areas)
            xx1 = jnp.maximum(x1j, x1s[...])
            yy1 = jnp.maximum(y1j, y1s[...])
            xx2 = jnp.minimum(x2j, x2s[...])
            yy2 = jnp.minimum(y2j, y2s[...])
            inter = jnp.maximum(xx2 - xx1 + 1.0, 0.0) * jnp.maximum(yy2 - yy1 + 1.0, 0.0)
            iou = inter / (aj + areas[...] - inter)
            css[...] = jnp.where((iou > NMS_THRESH) | (flat == j), -jnp.inf, cs)
            row8 = (jnp.where(li1 == 0, bf, 0.0) + jnp.where(li1 == 1, x1j, 0.0)
                    + jnp.where(li1 == 2, y1j, 0.0) + jnp.where(li1 == 3, x2j, 0.0)
                    + jnp.where(li1 == 4, y2j, 0.0))
            out_ref[pl.ds(step, 1), :] = row8

        return step + 1, alive

    lax.while_loop(cond, body, (jnp.int32(0), jnp.bool_(True)))


def _run(rpn_cls_scores, rpn_bbox_adjusts, img_shapes):
    B = rpn_cls_scores.shape[0]
    s = jnp.transpose(rpn_cls_scores[:, A:, :, :], (0, 2, 3, 1)).reshape(B * ROWS, 128)
    dxp = jnp.transpose(rpn_bbox_adjusts[:, 0::4], (0, 2, 3, 1)).reshape(B * ROWS, 128)
    dyp = jnp.transpose(rpn_bbox_adjusts[:, 1::4], (0, 2, 3, 1)).reshape(B * ROWS, 128)
    dwp = jnp.transpose(rpn_bbox_adjusts[:, 2::4], (0, 2, 3, 1)).reshape(B * ROWS, 128)
    dhp = jnp.transpose(rpn_bbox_adjusts[:, 3::4], (0, 2, 3, 1)).reshape(B * ROWS, 128)
    img = img_shapes.astype(jnp.float32).reshape(B, 1, 2)
    plane = lambda b: (b, 0)
    fixed = lambda b: (0, 0)
    out = pl.pallas_call(
        _proposal_body,
        grid=(B,),
        in_specs=[
            pl.BlockSpec((1, 1, 2), lambda b: (b, 0, 0), memory_space=pltpu.SMEM),
            pl.BlockSpec((ROWS, 128), plane),
            pl.BlockSpec((ROWS, 128), plane),
            pl.BlockSpec((ROWS, 128), plane),
            pl.BlockSpec((ROWS, 128), plane),
            pl.BlockSpec((ROWS, 128), plane),
            pl.BlockSpec((ROWS, 128), fixed),
            pl.BlockSpec((ROWS, 128), fixed),
            pl.BlockSpec((ROWS, 128), fixed),
            pl.BlockSpec((ROWS, 128), fixed),
        ],
        out_specs=pl.BlockSpec((OUT_ROWS, 8), plane),
        out_shape=jax.ShapeDtypeStruct((B * OUT_ROWS, 8), jnp.float32),
        scratch_shapes=[pltpu.VMEM((ROWS, 128), jnp.float32)] * 6,
        interpret=_INTERPRET,
    )(img, s, dxp, dyp, dwp, dhp,
      jnp.asarray(_WA), jnp.asarray(_HA), jnp.asarray(_CXA), jnp.asarray(_CYA))
    return out.reshape(B, OUT_ROWS, 8)[:, :POST_NMS_TOP_N, :5]


def kernel(rpn_cls_scores, rpn_bbox_adjusts, img_shapes, train):
    del train
    return _run(rpn_cls_scores, rpn_bbox_adjusts, img_shapes)
